# Initial kernel scaffold; baseline (speedup 1.0000x reference)
import jax
import jax.numpy as jnp
from jax.experimental import pallas as pl


def kernel(x, edge_index, question_symbols, ranking_difference, segment_ids, We, Wn, bn, Wc, bc):
    raise NotImplementedError("write your pallas kernel here")


# speedup vs baseline: 3.3443x; 3.3443x over previous
import functools

import jax
import jax.numpy as jnp
from jax import lax
from jax.experimental import pallas as pl
from jax.experimental.pallas import tpu as pltpu
from jax.experimental.pallas import tpu_sc as plsc

N = 50000
E = 800000
D = 64
L = 4
Q = 200000
S = 10000

NC = 2
NS = 16
NW = NC * NS

RT = 1568
NPAD = NW * RT
CAP = 65536
CE = 1600
STG = 3216
C = 128
QPT = 6400
QPAD = NW * QPT
CQ = 1600
SPAD = 10240
SCH = SPAD // NW


_CP = pltpu.CompilerParams(
    needs_layout_passes=False, use_tc_tiling_on_sc=False)


def _mesh():
  return plsc.VectorSubcoreMesh(
      core_axis_name="c", subcore_axis_name="s", num_cores=NC, num_subcores=NS)


def _wid():
  return lax.axis_index("s") * NC + lax.axis_index("c")


def _al8(v):
  return pl.multiple_of(v, 8)


def _bucket_edges(src, dst):
  nchunks = E // CE

  @functools.partial(
      pl.kernel,
      out_type=(jax.ShapeDtypeStruct((NW * CAP,), jnp.int32),
                jax.ShapeDtypeStruct((NW * CAP,), jnp.int32),
                jax.ShapeDtypeStruct((NW * 16,), jnp.int32)),
      mesh=_mesh(),
      compiler_params=_CP,
      scratch_types=[
          pltpu.VMEM((CE,), jnp.int32),
          pltpu.VMEM((CE,), jnp.int32),
          pltpu.VMEM((STG,), jnp.int32),
          pltpu.VMEM((STG,), jnp.int32),
          pltpu.VMEM((16,), jnp.int32),
      ])
  def k(src_h, dst_h, bsrc_h, bdst_h, cnt_h, sbuf, dbuf, st_s, st_d, cbuf):
    w = _wid()
    lo = w * RT
    hi = lo + RT

    def chunk(j, carry):
      cnt, flushed = carry
      pltpu.sync_copy(src_h.at[pl.ds(_al8(j * CE), CE)], sbuf)
      pltpu.sync_copy(dst_h.at[pl.ds(_al8(j * CE), CE)], dbuf)

      def grp(g, cnt):
        s16 = sbuf[pl.ds(g * 16, 16)]
        d16 = dbuf[pl.ds(g * 16, 16)]
        m = (d16 >= lo) & (d16 < hi)
        mi = m.astype(jnp.int32)
        pos = jnp.maximum(cnt + plsc.cumsum(mi) - 1, 0)
        plsc.store_scatter(st_s, [pos], s16, mask=m)
        plsc.store_scatter(st_d, [pos], d16 - lo, mask=m)
        return cnt + jnp.sum(mi)

      cnt = lax.fori_loop(0, CE // 16, grp, cnt)

      do_fl = cnt >= CE

      @pl.when(do_fl)
      def _():
        pltpu.sync_copy(st_s.at[pl.ds(0, CE)],
                        bsrc_h.at[pl.ds(_al8(w * CAP + flushed), CE)])
        pltpu.sync_copy(st_d.at[pl.ds(0, CE)],
                        bdst_h.at[pl.ds(_al8(w * CAP + flushed), CE)])
        for i in range(101):
          st_s[pl.ds(i * 16, 16)] = st_s[pl.ds(CE + i * 16, 16)]
          st_d[pl.ds(i * 16, 16)] = st_d[pl.ds(CE + i * 16, 16)]

      cnt = jnp.where(do_fl, cnt - CE, cnt)
      flushed = jnp.where(do_fl, flushed + CE, flushed)
      return cnt, flushed

    cnt, flushed = lax.fori_loop(0, nchunks, chunk,
                                 (jnp.int32(0), jnp.int32(0)))

    dummy_d = jnp.full((16,), RT, jnp.int32)
    zeros16 = jnp.zeros((16,), jnp.int32)
    for i in range(8):
      st_s[pl.ds(cnt + i * 16, 16)] = zeros16
      st_d[pl.ds(cnt + i * 16, 16)] = dummy_d
    cnt_pad = (cnt + (C - 1)) & ~(C - 1)
    nfl = cnt_pad >> 7

    def fl(jj, _):
      pltpu.sync_copy(st_s.at[pl.ds(jj * C, C)],
                      bsrc_h.at[pl.ds(_al8(w * CAP + flushed + jj * C), C)])
      pltpu.sync_copy(st_d.at[pl.ds(jj * C, C)],
                      bdst_h.at[pl.ds(_al8(w * CAP + flushed + jj * C), C)])
      return 0

    lax.fori_loop(0, nfl, fl, 0)
    cbuf[...] = zeros16 + (flushed + cnt_pad)
    pltpu.sync_copy(cbuf, cnt_h.at[pl.ds(_al8(w * 16), 16)])

  return k(src, dst)


def _aggregate(hw, bsrc, bdst, cnts):
  @functools.partial(
      pl.kernel,
      out_type=jax.ShapeDtypeStruct((NPAD, D), jnp.float32),
      mesh=_mesh(),
      compiler_params=_CP,
      scratch_types=[
          pltpu.VMEM((RT + 1, D), jnp.float32),
          pltpu.VMEM((C, D), jnp.float32),
          pltpu.VMEM((C,), jnp.int32),
          pltpu.VMEM((C,), jnp.int32),
          pltpu.VMEM((16,), jnp.int32),
      ])
  def k(hw_h, bsrc_h, bdst_h, cnt_h, agg_h, agg, stage, idxb, dstb, cntb):
    w = _wid()
    pltpu.sync_copy(cnt_h.at[pl.ds(_al8(w * 16), 16)], cntb)
    nch = cntb[pl.ds(0, 16)][0] >> 7

    zf = jnp.zeros((16,), jnp.float32)

    def zero(r, _):
      for kk in range(D // 16):
        agg[r, pl.ds(kk * 16, 16)] = zf
      return 0

    lax.fori_loop(0, RT + 1, zero, 0)

    def chunk(j, _):
      pltpu.sync_copy(bsrc_h.at[pl.ds(_al8(w * CAP + j * C), C)], idxb)
      pltpu.sync_copy(bdst_h.at[pl.ds(_al8(w * CAP + j * C), C)], dstb)
      pltpu.sync_copy(hw_h.at[idxb], stage)

      def group(g, _):
        dvec = dstb[pl.ds(g * 16, 16)]
        for e in range(16):
          r = dvec[e]
          for kk in range(D // 16):
            v = stage[g * 16 + e, pl.ds(kk * 16, 16)]
            plsc.addupdate(agg.at[r, pl.ds(kk * 16, 16)], v)
        return 0

      lax.fori_loop(0, C // 16, group, 0)
      return 0

    lax.fori_loop(0, nch, chunk, 0)
    pltpu.sync_copy(agg.at[pl.ds(0, RT)], agg_h.at[pl.ds(_al8(w * RT), RT)])

  return k(hw, bsrc, bdst, cnts)


def _tc_pre(x, we0):
  def body(x_ref, w_ref, o_ref):
    o_ref[...] = jnp.dot(x_ref[...], w_ref[...],
                         preferred_element_type=jnp.float32)

  return pl.pallas_call(
      body,
      grid=(25,),
      in_specs=[
          pl.BlockSpec((2000, D), lambda i: (i, 0)),
          pl.BlockSpec((D, D), lambda i: (0, 0)),
      ],
      out_specs=pl.BlockSpec((2000, D), lambda i: (i, 0)),
      out_shape=jax.ShapeDtypeStruct((N, D), jnp.float32),
  )(x, we0)


def _tc_mid(agg, wn, bnv, we_next):
  def body(a_ref, wn_ref, bn_ref, we_ref, o_ref):
    t = jnp.dot(a_ref[...], wn_ref[...], preferred_element_type=jnp.float32)
    t = jnp.maximum(t + bn_ref[...], 0.0)
    o_ref[...] = jnp.dot(t, we_ref[...], preferred_element_type=jnp.float32)

  nb = NPAD // 6272
  return pl.pallas_call(
      body,
      grid=(nb,),
      in_specs=[
          pl.BlockSpec((6272, D), lambda i: (i, 0)),
          pl.BlockSpec((D, D), lambda i: (0, 0)),
          pl.BlockSpec((1, D), lambda i: (0, 0)),
          pl.BlockSpec((D, D), lambda i: (0, 0)),
      ],
      out_specs=pl.BlockSpec((6272, D), lambda i: (i, 0)),
      out_shape=jax.ShapeDtypeStruct((NPAD, D), jnp.float32),
  )(agg, wn, bnv, we_next)


def _tc_fin(agg, wn, bnv, wc_pad, bcv):
  def body(a_ref, wn_ref, bn_ref, wc_ref, bc_ref, o_ref):
    t = jnp.dot(a_ref[...], wn_ref[...], preferred_element_type=jnp.float32)
    t = jnp.maximum(t + bn_ref[...], 0.0)
    o_ref[...] = jnp.dot(t, wc_ref[...],
                         preferred_element_type=jnp.float32) + bc_ref[...]

  nb = NPAD // 6272
  return pl.pallas_call(
      body,
      grid=(nb,),
      in_specs=[
          pl.BlockSpec((6272, D), lambda i: (i, 0)),
          pl.BlockSpec((D, D), lambda i: (0, 0)),
          pl.BlockSpec((1, D), lambda i: (0, 0)),
          pl.BlockSpec((D, 8), lambda i: (0, 0)),
          pl.BlockSpec((1, 8), lambda i: (0, 0)),
      ],
      out_specs=pl.BlockSpec((6272, 8), lambda i: (i, 0)),
      out_shape=jax.ShapeDtypeStruct((NPAD, 8), jnp.float32),
  )(agg, wn, bnv, wc_pad, bcv)


def _question_partials(cost, qs_p, rd_p, seg_p):
  @functools.partial(
      pl.kernel,
      out_type=jax.ShapeDtypeStruct((NW * SPAD,), jnp.float32),
      mesh=_mesh(),
      compiler_params=_CP,
      scratch_types=[
          pltpu.VMEM((NPAD,), jnp.float32),
          pltpu.VMEM((SPAD,), jnp.float32),
          pltpu.VMEM((SPAD,), jnp.float32),
          pltpu.VMEM((CQ + 16,), jnp.int32),
          pltpu.VMEM((CQ,), jnp.int32),
          pltpu.VMEM((CQ,), jnp.float32),
      ])
  def k(cost_h, qs_h, rd_h, seg_h, part_h, cost, se, ss, segb, qb, rb):
    w = _wid()
    base = w * QPT
    pltpu.sync_copy(cost_h, cost)

    zf = jnp.zeros((16,), jnp.float32)

    def zero(i, _):
      se[pl.ds(i * 16, 16)] = zf
      ss[pl.ds(i * 16, 16)] = zf
      return 0

    lax.fori_loop(0, SPAD // 16, zero, 0)

    iota = lax.iota(jnp.int32, 16)
    last_lane = iota == 15

    def chunk(j, carry):
      cbase = base + j * CQ
      pltpu.sync_copy(seg_h.at[pl.ds(_al8(cbase), CQ + 16)], segb)
      pltpu.sync_copy(qs_h.at[pl.ds(_al8(cbase), CQ)], qb)
      pltpu.sync_copy(rd_h.at[pl.ds(_al8(cbase), CQ)], rb)
      is_last_chunk = j == (QPT // CQ - 1)

      def grp(g, carry):
        prev_last, csum = carry
        seg = segb[pl.ds(g * 16, 16)]
        seg_n = segb[pl.ds(g * 16 + 16, 16)]
        pv = seg.at[jnp.maximum(iota - 1, 0)].get(mode="promise_in_bounds")
        pv = jnp.where(iota == 0, prev_last, pv)
        nx = seg.at[jnp.minimum(iota + 1, 15)].get(mode="promise_in_bounds")
        nfirst = seg_n[0]
        nx = jnp.where(last_lane, nfirst, nx)
        m_s = seg != pv
        m_e = seg != nx
        force = is_last_chunk & (g == (CQ // 16 - 1))
        m_e = m_e | (last_lane & force)

        qs16 = qb[pl.ds(g * 16, 16)]
        rd16 = rb[pl.ds(g * 16, 16)]
        cv = plsc.load_gather(cost, [qs16])
        pot = cv * rd16
        ics = plsc.cumsum(pot) + csum
        plsc.store_scatter(se, [seg], ics, mask=m_e)
        plsc.store_scatter(ss, [seg], ics - pot, mask=m_s)
        new_prev = seg[15]
        return new_prev, csum + jnp.sum(pot)

      return lax.fori_loop(0, CQ // 16, grp, carry)

    lax.fori_loop(0, QPT // CQ, chunk, (jnp.int32(-1), jnp.float32(0.0)))

    def fin(i, _):
      se[pl.ds(i * 16, 16)] = se[pl.ds(i * 16, 16)] - ss[pl.ds(i * 16, 16)]
      return 0

    lax.fori_loop(0, SPAD // 16, fin, 0)
    pltpu.sync_copy(se, part_h.at[pl.ds(_al8(w * SPAD), SPAD)])

  return k(cost, qs_p, rd_p, seg_p)


def _reduce_partials(part):
  @functools.partial(
      pl.kernel,
      out_type=jax.ShapeDtypeStruct((SPAD,), jnp.float32),
      mesh=_mesh(),
      compiler_params=_CP,
      scratch_types=[
          pltpu.VMEM((SCH,), jnp.float32),
          pltpu.VMEM((SCH,), jnp.float32),
      ])
  def k(part_h, out_h, acc, tmp):
    w = _wid()
    zf = jnp.zeros((16,), jnp.float32)
    for i in range(SCH // 16):
      acc[pl.ds(i * 16, 16)] = zf

    def row(r, _):
      pltpu.sync_copy(part_h.at[pl.ds(_al8(r * SPAD + w * SCH), SCH)], tmp)
      for i in range(SCH // 16):
        acc[pl.ds(i * 16, 16)] = acc[pl.ds(i * 16, 16)] + tmp[pl.ds(i * 16, 16)]
      return 0

    lax.fori_loop(0, NW, row, 0)
    pltpu.sync_copy(acc, out_h.at[pl.ds(_al8(w * SCH), SCH)])

  return k(part)


def kernel(x, edge_index, question_symbols, ranking_difference, segment_ids,
           We, Wn, bn, Wc, bc):
  src = edge_index[0]
  dst = edge_index[1]

  bsrc, bdst, cnts = _bucket_edges(src, dst)

  hw = _tc_pre(x, We[0])
  costp = None
  for l in range(L):
    agg = _aggregate(hw, bsrc, bdst, cnts)
    if l < L - 1:
      hw = _tc_mid(agg, Wn[l], bn[l][None], We[l + 1])
    else:
      wc_pad = jnp.pad(Wc, ((0, 0), (0, 7)))
      bcv = jnp.broadcast_to(bc[0], (1, 8)).astype(jnp.float32)
      costp = _tc_fin(agg, Wn[l], bn[l][None], wc_pad, bcv)

  cost = costp[:, 0]

  i32 = jnp.int32
  f32 = jnp.float32
  qs_p = jnp.concatenate(
      [question_symbols.astype(i32), jnp.zeros((QPAD - Q,), i32)])
  rd_p = jnp.concatenate([ranking_difference, jnp.zeros((QPAD - Q,), f32)])
  seg_p = jnp.concatenate(
      [segment_ids.astype(i32), jnp.full((QPAD - Q + 16,), S, i32)])

  part = _question_partials(cost, qs_p, rd_p, seg_p)
  logit_pad = _reduce_partials(part)
  return logit_pad[:S]


# speedup vs baseline: 5.1537x; 1.5411x over previous
import functools

import jax
import jax.numpy as jnp
from jax import lax
from jax.experimental import pallas as pl
from jax.experimental.pallas import tpu as pltpu
from jax.experimental.pallas import tpu_sc as plsc

N = 50000
E = 800000
D = 64
L = 4
Q = 200000
S = 10000

NC = 2
NS = 16
NW = NC * NS

RT = 1568
NPAD = NW * RT
CAP = 65536
CE = 1600
STG = 3216
C = 128
QPT = 6400
QPAD = NW * QPT
CQ = 1600
SPAD = 10240
SCH = SPAD // NW


_CP = pltpu.CompilerParams(
    needs_layout_passes=False, use_tc_tiling_on_sc=False)


def _mesh():
  return plsc.VectorSubcoreMesh(
      core_axis_name="c", subcore_axis_name="s", num_cores=NC, num_subcores=NS)


def _wid():
  return lax.axis_index("s") * NC + lax.axis_index("c")


def _al8(v):
  return pl.multiple_of(v, 8)


def _bucket_edges(src, dst):
  nchunks = E // CE

  @functools.partial(
      pl.kernel,
      out_type=(jax.ShapeDtypeStruct((NW * CAP,), jnp.int32),
                jax.ShapeDtypeStruct((NW * CAP,), jnp.int32),
                jax.ShapeDtypeStruct((NW * 16,), jnp.int32)),
      mesh=_mesh(),
      compiler_params=_CP,
      scratch_types=[
          pltpu.VMEM((2 * CE,), jnp.int32),
          pltpu.VMEM((2 * CE,), jnp.int32),
          pltpu.VMEM((STG,), jnp.int32),
          pltpu.VMEM((STG,), jnp.int32),
          pltpu.VMEM((16,), jnp.int32),
          pltpu.SemaphoreType.DMA,
          pltpu.SemaphoreType.DMA,
          pltpu.SemaphoreType.DMA,
          pltpu.SemaphoreType.DMA,
      ])
  def k(src_h, dst_h, bsrc_h, bdst_h, cnt_h, sbuf, dbuf, st_s, st_d, cbuf,
        ss0, ss1, sd0, sd1):
    w = _wid()
    lo = w * RT
    hi = lo + RT
    sems_s = (ss0, ss1)
    sems_d = (sd0, sd1)

    def start_in(j, bb):
      pltpu.async_copy(src_h.at[pl.ds(_al8(j * CE), CE)],
                       sbuf.at[pl.ds(bb * CE, CE)], sems_s[bb])
      pltpu.async_copy(dst_h.at[pl.ds(_al8(j * CE), CE)],
                       dbuf.at[pl.ds(bb * CE, CE)], sems_d[bb])

    def wait_in(j, bb):
      pltpu.make_async_copy(src_h.at[pl.ds(_al8(j * CE), CE)],
                            sbuf.at[pl.ds(bb * CE, CE)], sems_s[bb]).wait()
      pltpu.make_async_copy(dst_h.at[pl.ds(_al8(j * CE), CE)],
                            dbuf.at[pl.ds(bb * CE, CE)], sems_d[bb]).wait()

    start_in(0, 0)
    start_in(1, 1)

    def chunk(j, carry):
      cnt, flushed = carry
      b = j & 1
      for bb in range(2):
        @pl.when(b == bb)
        def _():
          wait_in(j, bb)

      boff = b * CE

      def grp(g, cnt):
        s16 = sbuf[pl.ds(boff + g * 16, 16)]
        d16 = dbuf[pl.ds(boff + g * 16, 16)]
        m = (d16 >= lo) & (d16 < hi)
        mi = m.astype(jnp.int32)
        pos = jnp.maximum(cnt + plsc.cumsum(mi) - 1, 0)
        plsc.store_scatter(st_s, [pos], s16, mask=m)
        plsc.store_scatter(st_d, [pos], d16 - lo, mask=m)
        return cnt + jnp.sum(mi)

      cnt = lax.fori_loop(0, CE // 16, grp, cnt)

      for bb in range(2):
        @pl.when((b == bb) & (j + 2 < E // CE))
        def _():
          start_in(j + 2, bb)

      do_fl = cnt >= CE

      @pl.when(do_fl)
      def _():
        pltpu.sync_copy(st_s.at[pl.ds(0, CE)],
                        bsrc_h.at[pl.ds(_al8(w * CAP + flushed), CE)])
        pltpu.sync_copy(st_d.at[pl.ds(0, CE)],
                        bdst_h.at[pl.ds(_al8(w * CAP + flushed), CE)])
        for i in range(101):
          st_s[pl.ds(i * 16, 16)] = st_s[pl.ds(CE + i * 16, 16)]
          st_d[pl.ds(i * 16, 16)] = st_d[pl.ds(CE + i * 16, 16)]

      cnt = jnp.where(do_fl, cnt - CE, cnt)
      flushed = jnp.where(do_fl, flushed + CE, flushed)
      return cnt, flushed

    cnt, flushed = lax.fori_loop(0, nchunks, chunk,
                                 (jnp.int32(0), jnp.int32(0)))

    dummy_d = jnp.full((16,), RT, jnp.int32)
    zeros16 = jnp.zeros((16,), jnp.int32)
    for i in range(8):
      st_s[pl.ds(cnt + i * 16, 16)] = zeros16
      st_d[pl.ds(cnt + i * 16, 16)] = dummy_d
    cnt_pad = (cnt + (C - 1)) & ~(C - 1)
    nfl = cnt_pad >> 7

    def fl(jj, _):
      pltpu.sync_copy(st_s.at[pl.ds(jj * C, C)],
                      bsrc_h.at[pl.ds(_al8(w * CAP + flushed + jj * C), C)])
      pltpu.sync_copy(st_d.at[pl.ds(jj * C, C)],
                      bdst_h.at[pl.ds(_al8(w * CAP + flushed + jj * C), C)])
      return 0

    lax.fori_loop(0, nfl, fl, 0)
    cbuf[...] = zeros16 + (flushed + cnt_pad)
    pltpu.sync_copy(cbuf, cnt_h.at[pl.ds(_al8(w * 16), 16)])

  return k(src, dst)


def _aggregate(hw, bsrc, bdst, cnts):
  @functools.partial(
      pl.kernel,
      out_type=jax.ShapeDtypeStruct((NPAD, D), jnp.float32),
      mesh=_mesh(),
      compiler_params=_CP,
      scratch_types=[
          pltpu.VMEM((RT + 1, D), jnp.float32),
          pltpu.VMEM((2 * C, D), jnp.float32),
          pltpu.VMEM((2 * C,), jnp.int32),
          pltpu.VMEM((2 * C,), jnp.int32),
          pltpu.VMEM((16,), jnp.int32),
          pltpu.SemaphoreType.DMA,
          pltpu.SemaphoreType.DMA,
          pltpu.SemaphoreType.DMA,
          pltpu.SemaphoreType.DMA,
          pltpu.SemaphoreType.DMA,
          pltpu.SemaphoreType.DMA,
      ])
  def k(hw_h, bsrc_h, bdst_h, cnt_h, agg_h, agg, stage, idxb, dstb, cntb,
        si0, si1, sd0, sd1, sg0, sg1):
    w = _wid()
    pltpu.sync_copy(cnt_h.at[pl.ds(_al8(w * 16), 16)], cntb)
    nch = cntb[pl.ds(0, 16)][0] >> 7
    sems_i = (si0, si1)
    sems_d = (sd0, sd1)
    sems_g = (sg0, sg1)

    def idx_src(j):
      return bsrc_h.at[pl.ds(_al8(w * CAP + j * C), C)]

    def dst_src(j):
      return bdst_h.at[pl.ds(_al8(w * CAP + j * C), C)]

    def start_in(j, bb):
      pltpu.async_copy(idx_src(j), idxb.at[pl.ds(bb * C, C)], sems_i[bb])
      pltpu.async_copy(dst_src(j), dstb.at[pl.ds(bb * C, C)], sems_d[bb])

    def start_gather(j, bb):
      pltpu.make_async_copy(idx_src(j), idxb.at[pl.ds(bb * C, C)],
                            sems_i[bb]).wait()
      pltpu.async_copy(hw_h.at[idxb.at[pl.ds(bb * C, C)]],
                       stage.at[pl.ds(bb * C, C)], sems_g[bb])

    def wait_gather(j, bb):
      pltpu.make_async_copy(hw_h.at[idxb.at[pl.ds(bb * C, C)]],
                            stage.at[pl.ds(bb * C, C)], sems_g[bb]).wait()
      pltpu.make_async_copy(dst_src(j), dstb.at[pl.ds(bb * C, C)],
                            sems_d[bb]).wait()

    @pl.when(nch > 0)
    def _():
      start_in(0, 0)

    @pl.when(nch > 1)
    def _():
      start_in(1, 1)

    @pl.when(nch > 0)
    def _():
      start_gather(0, 0)

    zf = jnp.zeros((16,), jnp.float32)

    def zero(r, _):
      for kk in range(D // 16):
        agg[r, pl.ds(kk * 16, 16)] = zf
      return 0

    lax.fori_loop(0, RT + 1, zero, 0)

    def chunk(j, _):
      b = j & 1
      for bb in range(2):
        @pl.when(b == bb)
        def _():
          @pl.when(j + 1 < nch)
          def _():
            start_gather(j + 1, 1 - bb)

          wait_gather(j, bb)

      boff = b * C

      def group(g, _):
        dvec = dstb[pl.ds(boff + g * 16, 16)]
        for e in range(16):
          r = dvec[e]
          for kk in range(D // 16):
            v = stage[boff + g * 16 + e, pl.ds(kk * 16, 16)]
            plsc.addupdate(agg.at[r, pl.ds(kk * 16, 16)], v)
        return 0

      lax.fori_loop(0, C // 16, group, 0)

      for bb in range(2):
        @pl.when((b == bb) & (j + 2 < nch))
        def _():
          start_in(j + 2, bb)

      return 0

    lax.fori_loop(0, nch, chunk, 0)
    pltpu.sync_copy(agg.at[pl.ds(0, RT)], agg_h.at[pl.ds(_al8(w * RT), RT)])

  return k(hw, bsrc, bdst, cnts)


def _tc_pre(x, we0):
  def body(x_ref, w_ref, o_ref):
    o_ref[...] = jnp.dot(x_ref[...], w_ref[...],
                         preferred_element_type=jnp.float32)

  return pl.pallas_call(
      body,
      grid=(25,),
      in_specs=[
          pl.BlockSpec((2000, D), lambda i: (i, 0)),
          pl.BlockSpec((D, D), lambda i: (0, 0)),
      ],
      out_specs=pl.BlockSpec((2000, D), lambda i: (i, 0)),
      out_shape=jax.ShapeDtypeStruct((N, D), jnp.float32),
  )(x, we0)


def _tc_mid(agg, wn, bnv, we_next):
  def body(a_ref, wn_ref, bn_ref, we_ref, o_ref):
    t = jnp.dot(a_ref[...], wn_ref[...], preferred_element_type=jnp.float32)
    t = jnp.maximum(t + bn_ref[...], 0.0)
    o_ref[...] = jnp.dot(t, we_ref[...], preferred_element_type=jnp.float32)

  nb = NPAD // 6272
  return pl.pallas_call(
      body,
      grid=(nb,),
      in_specs=[
          pl.BlockSpec((6272, D), lambda i: (i, 0)),
          pl.BlockSpec((D, D), lambda i: (0, 0)),
          pl.BlockSpec((1, D), lambda i: (0, 0)),
          pl.BlockSpec((D, D), lambda i: (0, 0)),
      ],
      out_specs=pl.BlockSpec((6272, D), lambda i: (i, 0)),
      out_shape=jax.ShapeDtypeStruct((NPAD, D), jnp.float32),
  )(agg, wn, bnv, we_next)


def _tc_fin(agg, wn, bnv, wc_pad, bcv):
  def body(a_ref, wn_ref, bn_ref, wc_ref, bc_ref, o_ref):
    t = jnp.dot(a_ref[...], wn_ref[...], preferred_element_type=jnp.float32)
    t = jnp.maximum(t + bn_ref[...], 0.0)
    o_ref[...] = jnp.dot(t, wc_ref[...],
                         preferred_element_type=jnp.float32) + bc_ref[...]

  nb = NPAD // 6272
  return pl.pallas_call(
      body,
      grid=(nb,),
      in_specs=[
          pl.BlockSpec((6272, D), lambda i: (i, 0)),
          pl.BlockSpec((D, D), lambda i: (0, 0)),
          pl.BlockSpec((1, D), lambda i: (0, 0)),
          pl.BlockSpec((D, 8), lambda i: (0, 0)),
          pl.BlockSpec((1, 8), lambda i: (0, 0)),
      ],
      out_specs=pl.BlockSpec((6272, 8), lambda i: (i, 0)),
      out_shape=jax.ShapeDtypeStruct((NPAD, 8), jnp.float32),
  )(agg, wn, bnv, wc_pad, bcv)


def _question_partials(cost, qs_p, rd_p, seg_p):
  @functools.partial(
      pl.kernel,
      out_type=jax.ShapeDtypeStruct((NW * SPAD,), jnp.float32),
      mesh=_mesh(),
      compiler_params=_CP,
      scratch_types=[
          pltpu.VMEM((NPAD,), jnp.float32),
          pltpu.VMEM((SPAD,), jnp.float32),
          pltpu.VMEM((SPAD,), jnp.float32),
          pltpu.VMEM((CQ + 16,), jnp.int32),
          pltpu.VMEM((CQ,), jnp.int32),
          pltpu.VMEM((CQ,), jnp.float32),
      ])
  def k(cost_h, qs_h, rd_h, seg_h, part_h, cost, se, ss, segb, qb, rb):
    w = _wid()
    base = w * QPT
    pltpu.sync_copy(cost_h, cost)

    zf = jnp.zeros((16,), jnp.float32)

    def zero(i, _):
      se[pl.ds(i * 16, 16)] = zf
      ss[pl.ds(i * 16, 16)] = zf
      return 0

    lax.fori_loop(0, SPAD // 16, zero, 0)

    iota = lax.iota(jnp.int32, 16)
    last_lane = iota == 15

    def chunk(j, carry):
      cbase = base + j * CQ
      pltpu.sync_copy(seg_h.at[pl.ds(_al8(cbase), CQ + 16)], segb)
      pltpu.sync_copy(qs_h.at[pl.ds(_al8(cbase), CQ)], qb)
      pltpu.sync_copy(rd_h.at[pl.ds(_al8(cbase), CQ)], rb)
      is_last_chunk = j == (QPT // CQ - 1)

      def grp(g, carry):
        prev_last, csum = carry
        seg = segb[pl.ds(g * 16, 16)]
        seg_n = segb[pl.ds(g * 16 + 16, 16)]
        pv = seg.at[jnp.maximum(iota - 1, 0)].get(mode="promise_in_bounds")
        pv = jnp.where(iota == 0, prev_last, pv)
        nx = seg.at[jnp.minimum(iota + 1, 15)].get(mode="promise_in_bounds")
        nfirst = seg_n[0]
        nx = jnp.where(last_lane, nfirst, nx)
        m_s = seg != pv
        m_e = seg != nx
        force = is_last_chunk & (g == (CQ // 16 - 1))
        m_e = m_e | (last_lane & force)

        qs16 = qb[pl.ds(g * 16, 16)]
        rd16 = rb[pl.ds(g * 16, 16)]
        cv = plsc.load_gather(cost, [qs16])
        pot = cv * rd16
        ics = plsc.cumsum(pot) + csum
        plsc.store_scatter(se, [seg], ics, mask=m_e)
        plsc.store_scatter(ss, [seg], ics - pot, mask=m_s)
        new_prev = seg[15]
        return new_prev, csum + jnp.sum(pot)

      return lax.fori_loop(0, CQ // 16, grp, carry)

    lax.fori_loop(0, QPT // CQ, chunk, (jnp.int32(-1), jnp.float32(0.0)))

    def fin(i, _):
      se[pl.ds(i * 16, 16)] = se[pl.ds(i * 16, 16)] - ss[pl.ds(i * 16, 16)]
      return 0

    lax.fori_loop(0, SPAD // 16, fin, 0)
    pltpu.sync_copy(se, part_h.at[pl.ds(_al8(w * SPAD), SPAD)])

  return k(cost, qs_p, rd_p, seg_p)


def _reduce_partials(part):
  @functools.partial(
      pl.kernel,
      out_type=jax.ShapeDtypeStruct((SPAD,), jnp.float32),
      mesh=_mesh(),
      compiler_params=_CP,
      scratch_types=[
          pltpu.VMEM((SCH,), jnp.float32),
          pltpu.VMEM((SCH,), jnp.float32),
      ])
  def k(part_h, out_h, acc, tmp):
    w = _wid()
    zf = jnp.zeros((16,), jnp.float32)
    for i in range(SCH // 16):
      acc[pl.ds(i * 16, 16)] = zf

    def row(r, _):
      pltpu.sync_copy(part_h.at[pl.ds(_al8(r * SPAD + w * SCH), SCH)], tmp)
      for i in range(SCH // 16):
        acc[pl.ds(i * 16, 16)] = acc[pl.ds(i * 16, 16)] + tmp[pl.ds(i * 16, 16)]
      return 0

    lax.fori_loop(0, NW, row, 0)
    pltpu.sync_copy(acc, out_h.at[pl.ds(_al8(w * SCH), SCH)])

  return k(part)


def kernel(x, edge_index, question_symbols, ranking_difference, segment_ids,
           We, Wn, bn, Wc, bc):
  src = edge_index[0]
  dst = edge_index[1]

  bsrc, bdst, cnts = _bucket_edges(src, dst)

  hw = _tc_pre(x, We[0])
  costp = None
  for l in range(L):
    agg = _aggregate(hw, bsrc, bdst, cnts)
    if l < L - 1:
      hw = _tc_mid(agg, Wn[l], bn[l][None], We[l + 1])
    else:
      wc_pad = jnp.pad(Wc, ((0, 0), (0, 7)))
      bcv = jnp.broadcast_to(bc[0], (1, 8)).astype(jnp.float32)
      costp = _tc_fin(agg, Wn[l], bn[l][None], wc_pad, bcv)

  cost = costp[:, 0]

  i32 = jnp.int32
  f32 = jnp.float32
  qs_p = jnp.concatenate(
      [question_symbols.astype(i32), jnp.zeros((QPAD - Q,), i32)])
  rd_p = jnp.concatenate([ranking_difference, jnp.zeros((QPAD - Q,), f32)])
  seg_p = jnp.concatenate(
      [segment_ids.astype(i32), jnp.full((QPAD - Q + 16,), S, i32)])

  part = _question_partials(cost, qs_p, rd_p, seg_p)
  logit_pad = _reduce_partials(part)
  return logit_pad[:S]


# speedup vs baseline: 9.1479x; 1.7750x over previous
import functools

import jax
import jax.numpy as jnp
from jax import lax
from jax.experimental import pallas as pl
from jax.experimental.pallas import tpu as pltpu
from jax.experimental.pallas import tpu_sc as plsc

N = 50000
E = 800000
D = 64
L = 4
Q = 200000
S = 10000

NC = 2
NS = 16
NW = NC * NS

RT = 1568
NPAD = NW * RT
CAP = 65536
CE = 1600
STG = 3216
C = 128
QPT = 6400
QPAD = NW * QPT
CQ = 1600
SPAD = 10240
SCH = SPAD // NW


_CP = pltpu.CompilerParams(
    needs_layout_passes=False, use_tc_tiling_on_sc=False)


def _mesh():
  return plsc.VectorSubcoreMesh(
      core_axis_name="c", subcore_axis_name="s", num_cores=NC, num_subcores=NS)


def _wid():
  return lax.axis_index("s") * NC + lax.axis_index("c")


def _al8(v):
  return pl.multiple_of(v, 8)


def _bucket_edges(src, dst):
  nchunks = E // CE

  @functools.partial(
      pl.kernel,
      out_type=(jax.ShapeDtypeStruct((NW * CAP,), jnp.int32),
                jax.ShapeDtypeStruct((NW * CAP,), jnp.int32),
                jax.ShapeDtypeStruct((NW * 16,), jnp.int32)),
      mesh=_mesh(),
      compiler_params=_CP,
      scratch_types=[
          pltpu.VMEM((2 * CE,), jnp.int32),
          pltpu.VMEM((2 * CE,), jnp.int32),
          pltpu.VMEM((STG,), jnp.int32),
          pltpu.VMEM((STG,), jnp.int32),
          pltpu.VMEM((16,), jnp.int32),
          pltpu.SemaphoreType.DMA,
          pltpu.SemaphoreType.DMA,
          pltpu.SemaphoreType.DMA,
          pltpu.SemaphoreType.DMA,
      ])
  def k(src_h, dst_h, bsrc_h, bdst_h, cnt_h, sbuf, dbuf, st_s, st_d, cbuf,
        ss0, ss1, sd0, sd1):
    w = _wid()
    lo = w * RT
    hi = lo + RT
    sems_s = (ss0, ss1)
    sems_d = (sd0, sd1)

    def start_in(j, bb):
      pltpu.async_copy(src_h.at[pl.ds(_al8(j * CE), CE)],
                       sbuf.at[pl.ds(bb * CE, CE)], sems_s[bb])
      pltpu.async_copy(dst_h.at[pl.ds(_al8(j * CE), CE)],
                       dbuf.at[pl.ds(bb * CE, CE)], sems_d[bb])

    def wait_in(j, bb):
      pltpu.make_async_copy(src_h.at[pl.ds(_al8(j * CE), CE)],
                            sbuf.at[pl.ds(bb * CE, CE)], sems_s[bb]).wait()
      pltpu.make_async_copy(dst_h.at[pl.ds(_al8(j * CE), CE)],
                            dbuf.at[pl.ds(bb * CE, CE)], sems_d[bb]).wait()

    start_in(0, 0)
    start_in(1, 1)

    def chunk(j, carry):
      cnt, flushed = carry
      b = j & 1
      for bb in range(2):
        @pl.when(b == bb)
        def _():
          wait_in(j, bb)

      boff = b * CE

      def grp(g, cnt):
        s16 = sbuf[pl.ds(boff + g * 16, 16)]
        d16 = dbuf[pl.ds(boff + g * 16, 16)]
        m = (d16 >= lo) & (d16 < hi)
        mi = m.astype(jnp.int32)
        pos = jnp.maximum(cnt + plsc.cumsum(mi) - 1, 0)
        plsc.store_scatter(st_s, [pos], s16, mask=m)
        plsc.store_scatter(st_d, [pos], d16 - lo, mask=m)
        return cnt + jnp.sum(mi)

      cnt = plsc.parallel_loop(0, CE // 16, unroll=2, carry=cnt)(grp)

      for bb in range(2):
        @pl.when((b == bb) & (j + 2 < E // CE))
        def _():
          start_in(j + 2, bb)

      do_fl = cnt >= CE

      @pl.when(do_fl)
      def _():
        pltpu.sync_copy(st_s.at[pl.ds(0, CE)],
                        bsrc_h.at[pl.ds(_al8(w * CAP + flushed), CE)])
        pltpu.sync_copy(st_d.at[pl.ds(0, CE)],
                        bdst_h.at[pl.ds(_al8(w * CAP + flushed), CE)])
        for i in range(101):
          st_s[pl.ds(i * 16, 16)] = st_s[pl.ds(CE + i * 16, 16)]
          st_d[pl.ds(i * 16, 16)] = st_d[pl.ds(CE + i * 16, 16)]

      cnt = jnp.where(do_fl, cnt - CE, cnt)
      flushed = jnp.where(do_fl, flushed + CE, flushed)
      return cnt, flushed

    cnt, flushed = lax.fori_loop(0, nchunks, chunk,
                                 (jnp.int32(0), jnp.int32(0)))

    dummy_d = jnp.full((16,), RT, jnp.int32)
    zeros16 = jnp.zeros((16,), jnp.int32)
    for i in range(8):
      st_s[pl.ds(cnt + i * 16, 16)] = zeros16
      st_d[pl.ds(cnt + i * 16, 16)] = dummy_d
    cnt_pad = (cnt + (C - 1)) & ~(C - 1)
    nfl = cnt_pad >> 7

    def fl(jj, _):
      pltpu.sync_copy(st_s.at[pl.ds(jj * C, C)],
                      bsrc_h.at[pl.ds(_al8(w * CAP + flushed + jj * C), C)])
      pltpu.sync_copy(st_d.at[pl.ds(jj * C, C)],
                      bdst_h.at[pl.ds(_al8(w * CAP + flushed + jj * C), C)])
      return 0

    lax.fori_loop(0, nfl, fl, 0)
    cbuf[...] = zeros16 + (flushed + cnt_pad)
    pltpu.sync_copy(cbuf, cnt_h.at[pl.ds(_al8(w * 16), 16)])

  return k(src, dst)


def _aggregate(hw, bsrc, bdst, cnts):
  @functools.partial(
      pl.kernel,
      out_type=jax.ShapeDtypeStruct((NPAD, D), jnp.float32),
      mesh=_mesh(),
      compiler_params=_CP,
      scratch_types=[
          pltpu.VMEM((RT + 1, D), jnp.float32),
          pltpu.VMEM((2 * C, D), jnp.float32),
          pltpu.VMEM((2 * C,), jnp.int32),
          pltpu.VMEM((2 * C,), jnp.int32),
          pltpu.VMEM((16,), jnp.int32),
          pltpu.SemaphoreType.DMA,
          pltpu.SemaphoreType.DMA,
          pltpu.SemaphoreType.DMA,
          pltpu.SemaphoreType.DMA,
          pltpu.SemaphoreType.DMA,
          pltpu.SemaphoreType.DMA,
      ])
  def k(hw_h, bsrc_h, bdst_h, cnt_h, agg_h, agg, stage, idxb, dstb, cntb,
        si0, si1, sd0, sd1, sg0, sg1):
    w = _wid()
    pltpu.sync_copy(cnt_h.at[pl.ds(_al8(w * 16), 16)], cntb)
    nch = cntb[pl.ds(0, 16)][0] >> 7
    sems_i = (si0, si1)
    sems_d = (sd0, sd1)
    sems_g = (sg0, sg1)

    def idx_src(j):
      return bsrc_h.at[pl.ds(_al8(w * CAP + j * C), C)]

    def dst_src(j):
      return bdst_h.at[pl.ds(_al8(w * CAP + j * C), C)]

    def start_in(j, bb):
      pltpu.async_copy(idx_src(j), idxb.at[pl.ds(bb * C, C)], sems_i[bb])
      pltpu.async_copy(dst_src(j), dstb.at[pl.ds(bb * C, C)], sems_d[bb])

    def start_gather(j, bb):
      pltpu.make_async_copy(idx_src(j), idxb.at[pl.ds(bb * C, C)],
                            sems_i[bb]).wait()
      pltpu.async_copy(hw_h.at[idxb.at[pl.ds(bb * C, C)]],
                       stage.at[pl.ds(bb * C, C)], sems_g[bb])

    def wait_gather(j, bb):
      pltpu.make_async_copy(hw_h.at[idxb.at[pl.ds(bb * C, C)]],
                            stage.at[pl.ds(bb * C, C)], sems_g[bb]).wait()
      pltpu.make_async_copy(dst_src(j), dstb.at[pl.ds(bb * C, C)],
                            sems_d[bb]).wait()

    @pl.when(nch > 0)
    def _():
      start_in(0, 0)

    @pl.when(nch > 1)
    def _():
      start_in(1, 1)

    @pl.when(nch > 0)
    def _():
      start_gather(0, 0)

    zf = jnp.zeros((16,), jnp.float32)

    def _zero(r):
      for kk in range(D // 16):
        agg[r, pl.ds(kk * 16, 16)] = zf

    plsc.parallel_loop(0, RT + 1, unroll=4)(_zero)

    def chunk(j, _):
      b = j & 1
      for bb in range(2):
        @pl.when(b == bb)
        def _():
          @pl.when(j + 1 < nch)
          def _():
            start_gather(j + 1, 1 - bb)

          wait_gather(j, bb)

      boff = b * C

      def _group(g):
        dvec = dstb[pl.ds(boff + g * 16, 16)]
        for e in range(16):
          r = dvec[e]
          for kk in range(D // 16):
            v = stage[boff + g * 16 + e, pl.ds(kk * 16, 16)]
            plsc.addupdate(agg.at[r, pl.ds(kk * 16, 16)], v)

      plsc.parallel_loop(0, C // 16, unroll=2)(_group)

      for bb in range(2):
        @pl.when((b == bb) & (j + 2 < nch))
        def _():
          start_in(j + 2, bb)

      return 0

    lax.fori_loop(0, nch, chunk, 0)
    pltpu.sync_copy(agg.at[pl.ds(0, RT)], agg_h.at[pl.ds(_al8(w * RT), RT)])

  return k(hw, bsrc, bdst, cnts)


def _tc_pre(x, we0):
  def body(x_ref, w_ref, o_ref):
    o_ref[...] = jnp.dot(x_ref[...], w_ref[...],
                         preferred_element_type=jnp.float32)

  return pl.pallas_call(
      body,
      grid=(25,),
      in_specs=[
          pl.BlockSpec((2000, D), lambda i: (i, 0)),
          pl.BlockSpec((D, D), lambda i: (0, 0)),
      ],
      out_specs=pl.BlockSpec((2000, D), lambda i: (i, 0)),
      out_shape=jax.ShapeDtypeStruct((N, D), jnp.float32),
  )(x, we0)


def _tc_mid(agg, wn, bnv, we_next):
  def body(a_ref, wn_ref, bn_ref, we_ref, o_ref):
    t = jnp.dot(a_ref[...], wn_ref[...], preferred_element_type=jnp.float32)
    t = jnp.maximum(t + bn_ref[...], 0.0)
    o_ref[...] = jnp.dot(t, we_ref[...], preferred_element_type=jnp.float32)

  nb = NPAD // 6272
  return pl.pallas_call(
      body,
      grid=(nb,),
      in_specs=[
          pl.BlockSpec((6272, D), lambda i: (i, 0)),
          pl.BlockSpec((D, D), lambda i: (0, 0)),
          pl.BlockSpec((1, D), lambda i: (0, 0)),
          pl.BlockSpec((D, D), lambda i: (0, 0)),
      ],
      out_specs=pl.BlockSpec((6272, D), lambda i: (i, 0)),
      out_shape=jax.ShapeDtypeStruct((NPAD, D), jnp.float32),
  )(agg, wn, bnv, we_next)


def _tc_fin(agg, wn, bnv, wc_pad, bcv):
  def body(a_ref, wn_ref, bn_ref, wc_ref, bc_ref, o_ref):
    t = jnp.dot(a_ref[...], wn_ref[...], preferred_element_type=jnp.float32)
    t = jnp.maximum(t + bn_ref[...], 0.0)
    o_ref[...] = jnp.dot(t, wc_ref[...],
                         preferred_element_type=jnp.float32) + bc_ref[...]

  nb = NPAD // 6272
  return pl.pallas_call(
      body,
      grid=(nb,),
      in_specs=[
          pl.BlockSpec((6272, D), lambda i: (i, 0)),
          pl.BlockSpec((D, D), lambda i: (0, 0)),
          pl.BlockSpec((1, D), lambda i: (0, 0)),
          pl.BlockSpec((D, 8), lambda i: (0, 0)),
          pl.BlockSpec((1, 8), lambda i: (0, 0)),
      ],
      out_specs=pl.BlockSpec((6272, 8), lambda i: (i, 0)),
      out_shape=jax.ShapeDtypeStruct((NPAD, 8), jnp.float32),
  )(agg, wn, bnv, wc_pad, bcv)


def _question_partials(cost, qs_p, rd_p, seg_p):
  @functools.partial(
      pl.kernel,
      out_type=jax.ShapeDtypeStruct((NW * SPAD,), jnp.float32),
      mesh=_mesh(),
      compiler_params=_CP,
      scratch_types=[
          pltpu.VMEM((NPAD,), jnp.float32),
          pltpu.VMEM((SPAD,), jnp.float32),
          pltpu.VMEM((SPAD,), jnp.float32),
          pltpu.VMEM((CQ + 16,), jnp.int32),
          pltpu.VMEM((CQ,), jnp.int32),
          pltpu.VMEM((CQ,), jnp.float32),
      ])
  def k(cost_h, qs_h, rd_h, seg_h, part_h, cost, se, ss, segb, qb, rb):
    w = _wid()
    base = w * QPT
    pltpu.sync_copy(cost_h, cost)

    zf = jnp.zeros((16,), jnp.float32)

    def zero(i, _):
      se[pl.ds(i * 16, 16)] = zf
      ss[pl.ds(i * 16, 16)] = zf
      return 0

    lax.fori_loop(0, SPAD // 16, zero, 0)

    iota = lax.iota(jnp.int32, 16)
    last_lane = iota == 15

    def chunk(j, carry):
      cbase = base + j * CQ
      pltpu.sync_copy(seg_h.at[pl.ds(_al8(cbase), CQ + 16)], segb)
      pltpu.sync_copy(qs_h.at[pl.ds(_al8(cbase), CQ)], qb)
      pltpu.sync_copy(rd_h.at[pl.ds(_al8(cbase), CQ)], rb)
      is_last_chunk = j == (QPT // CQ - 1)

      def grp(g, carry):
        prev_last, csum = carry
        seg = segb[pl.ds(g * 16, 16)]
        seg_n = segb[pl.ds(g * 16 + 16, 16)]
        pv = seg.at[jnp.maximum(iota - 1, 0)].get(mode="promise_in_bounds")
        pv = jnp.where(iota == 0, prev_last, pv)
        nx = seg.at[jnp.minimum(iota + 1, 15)].get(mode="promise_in_bounds")
        nfirst = seg_n[0]
        nx = jnp.where(last_lane, nfirst, nx)
        m_s = seg != pv
        m_e = seg != nx
        force = is_last_chunk & (g == (CQ // 16 - 1))
        m_e = m_e | (last_lane & force)

        qs16 = qb[pl.ds(g * 16, 16)]
        rd16 = rb[pl.ds(g * 16, 16)]
        cv = plsc.load_gather(cost, [qs16])
        pot = cv * rd16
        ics = plsc.cumsum(pot) + csum
        plsc.store_scatter(se, [seg], ics, mask=m_e)
        plsc.store_scatter(ss, [seg], ics - pot, mask=m_s)
        new_prev = seg[15]
        return new_prev, csum + jnp.sum(pot)

      return plsc.parallel_loop(0, CQ // 16, unroll=2, carry=carry)(grp)

    lax.fori_loop(0, QPT // CQ, chunk, (jnp.int32(-1), jnp.float32(0.0)))

    def fin(i, _):
      se[pl.ds(i * 16, 16)] = se[pl.ds(i * 16, 16)] - ss[pl.ds(i * 16, 16)]
      return 0

    lax.fori_loop(0, SPAD // 16, fin, 0)
    pltpu.sync_copy(se, part_h.at[pl.ds(_al8(w * SPAD), SPAD)])

  return k(cost, qs_p, rd_p, seg_p)


def _reduce_partials(part):
  @functools.partial(
      pl.kernel,
      out_type=jax.ShapeDtypeStruct((SPAD,), jnp.float32),
      mesh=_mesh(),
      compiler_params=_CP,
      scratch_types=[
          pltpu.VMEM((SCH,), jnp.float32),
          pltpu.VMEM((SCH,), jnp.float32),
      ])
  def k(part_h, out_h, acc, tmp):
    w = _wid()
    zf = jnp.zeros((16,), jnp.float32)
    for i in range(SCH // 16):
      acc[pl.ds(i * 16, 16)] = zf

    def row(r, _):
      pltpu.sync_copy(part_h.at[pl.ds(_al8(r * SPAD + w * SCH), SCH)], tmp)
      for i in range(SCH // 16):
        acc[pl.ds(i * 16, 16)] = acc[pl.ds(i * 16, 16)] + tmp[pl.ds(i * 16, 16)]
      return 0

    lax.fori_loop(0, NW, row, 0)
    pltpu.sync_copy(acc, out_h.at[pl.ds(_al8(w * SCH), SCH)])

  return k(part)


def kernel(x, edge_index, question_symbols, ranking_difference, segment_ids,
           We, Wn, bn, Wc, bc):
  src = edge_index[0]
  dst = edge_index[1]

  bsrc, bdst, cnts = _bucket_edges(src, dst)

  hw = _tc_pre(x, We[0])
  costp = None
  for l in range(L):
    agg = _aggregate(hw, bsrc, bdst, cnts)
    if l < L - 1:
      hw = _tc_mid(agg, Wn[l], bn[l][None], We[l + 1])
    else:
      wc_pad = jnp.pad(Wc, ((0, 0), (0, 7)))
      bcv = jnp.broadcast_to(bc[0], (1, 8)).astype(jnp.float32)
      costp = _tc_fin(agg, Wn[l], bn[l][None], wc_pad, bcv)

  cost = costp[:, 0]

  i32 = jnp.int32
  f32 = jnp.float32
  qs_p = jnp.concatenate(
      [question_symbols.astype(i32), jnp.zeros((QPAD - Q,), i32)])
  rd_p = jnp.concatenate([ranking_difference, jnp.zeros((QPAD - Q,), f32)])
  seg_p = jnp.concatenate(
      [segment_ids.astype(i32), jnp.full((QPAD - Q + 16,), S, i32)])

  part = _question_partials(cost, qs_p, rd_p, seg_p)
  logit_pad = _reduce_partials(part)
  return logit_pad[:S]
